# Initial kernel scaffold; baseline (speedup 1.0000x reference)
import jax
import jax.numpy as jnp
from jax.experimental import pallas as pl


def kernel(im):
    raise NotImplementedError("write your pallas kernel here")


# speedup vs baseline: 1.0312x; 1.0312x over previous
import functools

import jax
import jax.numpy as jnp
from jax import lax
from jax.experimental import pallas as pl
from jax.experimental.pallas import tpu as pltpu
from jax.experimental.pallas import tpu_sc as plsc

H = 224
W = 224
HP = H + 4
WP = W + 4
WB = WP
NCHUNK = WP // 16
NIMG = 16 * 96
NWORK = 32
PER_W = NIMG // NWORK
NPAIR = PER_W // 2

_MESH = plsc.VectorSubcoreMesh(
    core_axis_name="c", subcore_axis_name="s", num_cores=2, num_subcores=16
)


@functools.partial(
    pl.kernel,
    out_type=jax.ShapeDtypeStruct((NIMG, HP, WP), jnp.float32),
    mesh=_MESH,
    scratch_types=[
        pltpu.VMEM((H, WB), jnp.float32),
        pltpu.VMEM((H, WB), jnp.float32),
        pltpu.SemaphoreType.DMA,
        pltpu.SemaphoreType.DMA,
        pltpu.SemaphoreType.DMA,
        pltpu.SemaphoreType.DMA,
    ],
    compiler_params=pltpu.CompilerParams(
        use_tc_tiling_on_sc=False, needs_layout_passes=False
    ),
)
def _pad_kernel(im_hbm, out_hbm, b0, b1, si0, si1, so0, so1):
    wid = lax.axis_index("s") * 2 + lax.axis_index("c")
    base = wid * PER_W
    iota = lax.iota(jnp.int32, 16)

    def start_load(img, buf, sem):
        pltpu.make_async_copy(im_hbm.at[img], buf.at[:, pl.ds(0, W)], sem).start()

    def wait_load(buf, sem):
        pltpu.make_async_copy(im_hbm.at[0], buf.at[:, pl.ds(0, W)], sem).wait()

    def _store_descs(img, buf, sem):
        return (
            pltpu.make_async_copy(buf, out_hbm.at[img, pl.ds(2, H)], sem),
            pltpu.make_async_copy(
                buf.at[pl.ds(1, 1)], out_hbm.at[img, pl.ds(0, 1)], sem
            ),
            pltpu.make_async_copy(
                buf.at[pl.ds(0, 1)], out_hbm.at[img, pl.ds(1, 1)], sem
            ),
            pltpu.make_async_copy(
                buf.at[pl.ds(H - 1, 1)], out_hbm.at[img, pl.ds(HP - 2, 1)], sem
            ),
            pltpu.make_async_copy(
                buf.at[pl.ds(H - 2, 1)], out_hbm.at[img, pl.ds(HP - 1, 1)], sem
            ),
        )

    def start_stores(img, buf, sem):
        for c in _store_descs(img, buf, sem):
            c.start()

    def wait_stores(buf, sem):
        for c in _store_descs(0, buf, sem):
            c.wait()

    cols = []
    for k in range(NCHUNK + 1):
        x = iota + (16 * k - 2)
        x = jnp.where(x < 0, -1 - x, x)
        x = jnp.where(x > W - 1, 2 * W - 1 - x, x)
        cols.append(x)
    tail_dst = jnp.minimum(16 * NCHUNK + iota, WP - 1)
    tail_msk = iota < WP - 16 * NCHUNK

    def shift(buf):
        def row_body(r, carry):
            rv = jnp.full((16,), r, jnp.int32)
            v = plsc.load_gather(buf, [rv, cols[NCHUNK]], mask=tail_msk)
            plsc.store_scatter(buf, [rv, tail_dst], v, mask=tail_msk)
            for k in range(NCHUNK - 1, -1, -1):
                v = plsc.load_gather(buf, [rv, cols[k]])
                buf[r, pl.ds(16 * k, 16)] = v
            return carry

        lax.fori_loop(0, H, row_body, 0)

    start_load(base, b0, si0)

    def body(j, carry):
        i0 = base + 2 * j
        wait_load(b0, si0)
        pl.when(j > 0)(lambda: wait_stores(b1, so1))
        start_load(i0 + 1, b1, si1)
        shift(b0)
        start_stores(i0, b0, so0)
        wait_load(b1, si1)
        shift(b1)
        wait_stores(b0, so0)
        pl.when(j < NPAIR - 1)(lambda: start_load(i0 + 2, b0, si0))
        start_stores(i0 + 1, b1, so1)
        return carry

    lax.fori_loop(0, NPAIR, body, 0)
    wait_stores(b1, so1)


def kernel(im):
    b, c, h, w = im.shape
    out = _pad_kernel(im.reshape(b * c, h, w))
    return out.reshape(b, c, h + 4, w + 4)


# speedup vs baseline: 2.1087x; 2.0450x over previous
import functools

import jax
import jax.numpy as jnp
from jax import lax
from jax.experimental import pallas as pl
from jax.experimental.pallas import tpu as pltpu
from jax.experimental.pallas import tpu_sc as plsc

B = 16
C = 96
H = 224
W = 224
HP = H + 4
WP = W + 4
NIMG = B * C
NWORK = 32
PER_W = NIMG // NWORK

HT = 120
HB = HP - HT
IB = 112
NCHUNK = WP // 16

_MESH = plsc.VectorSubcoreMesh(
    core_axis_name="c", subcore_axis_name="s", num_cores=2, num_subcores=16
)


@functools.partial(
    pl.kernel,
    out_type=jax.ShapeDtypeStruct((B, C, HP, WP), jnp.float32),
    mesh=_MESH,
    scratch_types=[
        pltpu.VMEM((HT, W), jnp.float32),
        pltpu.VMEM((HT, W), jnp.float32),
        pltpu.VMEM((HT, WP), jnp.float32),
        pltpu.VMEM((HB, WP), jnp.float32),
        pltpu.SemaphoreType.DMA,
        pltpu.SemaphoreType.DMA,
        pltpu.SemaphoreType.DMA,
        pltpu.SemaphoreType.DMA,
    ],
    compiler_params=pltpu.CompilerParams(
        use_tc_tiling_on_sc=True, needs_layout_passes=False
    ),
)
def _pad_kernel(im_hbm, out_hbm, inT, inB, outT, outB, siT, siB, soT, soB):
    wid = lax.axis_index("s") * 2 + lax.axis_index("c")
    base = wid * PER_W
    iota = lax.iota(jnp.int32, 16)

    cols = []
    for k in range(NCHUNK + 1):
        x = iota + (16 * k - 2)
        x = jnp.where(x < 0, -1 - x, x)
        x = jnp.where(x > W - 1, 2 * W - 1 - x, x)
        cols.append(x)
    dsts = [iota + 16 * k for k in range(NCHUNK)]
    tail_dst = jnp.minimum(16 * NCHUNK + iota, WP - 1)
    tail_msk = iota < WP - 16 * NCHUNK

    def bc(img):
        return img // C, img % C

    def start_load_T(img, sem):
        b, c = bc(img)
        pltpu.make_async_copy(im_hbm.at[b, c, pl.ds(0, HT)], inT, sem).start()

    def wait_load_T(sem):
        pltpu.make_async_copy(im_hbm.at[0, 0, pl.ds(0, HT)], inT, sem).wait()

    def start_load_B(img, sem):
        b, c = bc(img)
        pltpu.make_async_copy(
            im_hbm.at[b, c, pl.ds(IB, H - IB)], inB.at[pl.ds(0, H - IB)], sem
        ).start()

    def wait_load_B(sem):
        pltpu.make_async_copy(
            im_hbm.at[0, 0, pl.ds(IB, H - IB)], inB.at[pl.ds(0, H - IB)], sem
        ).wait()

    def start_store_T(img, sem):
        b, c = bc(img)
        pltpu.make_async_copy(outT, out_hbm.at[b, c, pl.ds(0, HT)], sem).start()

    def wait_store_T(sem):
        pltpu.make_async_copy(outT, out_hbm.at[0, 0, pl.ds(0, HT)], sem).wait()

    def start_store_B(img, sem):
        b, c = bc(img)
        pltpu.make_async_copy(outB, out_hbm.at[b, c, pl.ds(HT, HB)], sem).start()

    def wait_store_B(sem):
        pltpu.make_async_copy(outB, out_hbm.at[0, 0, pl.ds(HT, HB)], sem).wait()

    def gather_band(src, dst, nrows, out_row0, in_row0):
        def row_body(rl, carry):
            r = rl + out_row0
            y = r - 2
            y = jnp.where(y < 0, -1 - y, y)
            y = jnp.where(y > H - 1, 2 * H - 1 - y, y)
            yv = jnp.full((16,), y - in_row0, jnp.int32)
            rv = jnp.full((16,), rl, jnp.int32)
            for k in range(NCHUNK):
                v = plsc.load_gather(src, [yv, cols[k]])
                plsc.store_scatter(dst, [rv, dsts[k]], v)
            v = plsc.load_gather(src, [yv, cols[NCHUNK]], mask=tail_msk)
            plsc.store_scatter(dst, [rv, tail_dst], v, mask=tail_msk)
            return carry

        lax.fori_loop(0, nrows, row_body, 0)

    start_load_T(base, siT)

    def body(j, carry):
        img = base + j
        wait_load_T(siT)
        pl.when(j > 0)(lambda: wait_store_B(soB))
        start_load_B(img, siB)
        gather_band(inT, outT, HT, 0, 0)
        start_store_T(img, soT)
        wait_load_B(siB)
        gather_band(inB, outB, HB, HT, IB)
        wait_store_T(soT)
        pl.when(j < PER_W - 1)(lambda: start_load_T(img + 1, siT))
        start_store_B(img, soB)
        return carry

    lax.fori_loop(0, PER_W, body, 0)
    wait_store_B(soB)


def kernel(im):
    return _pad_kernel(im)


# speedup vs baseline: 2.2543x; 1.0691x over previous
import functools

import jax
import jax.numpy as jnp
from jax import lax
from jax.experimental import pallas as pl
from jax.experimental.pallas import tpu as pltpu
from jax.experimental.pallas import tpu_sc as plsc

B = 16
C = 96
H = 224
W = 224
HP = H + 4
WP = W + 4
NWORK = 32
NCHUNK = WP // 16

NBAND = H // 8
NCB = C // 8
NA = B * NBAND * NCB
PER_WA = NA // NWORK
NB = B * NCB * 2
PER_WB = NB // NWORK

_MESH = plsc.VectorSubcoreMesh(
    core_axis_name="c", subcore_axis_name="s", num_cores=2, num_subcores=16
)


@functools.partial(
    pl.kernel,
    out_type=jax.ShapeDtypeStruct((B, HP, C, WP), jnp.float32),
    mesh=_MESH,
    scratch_types=[
        pltpu.VMEM((8, 8, W), jnp.float32),
        pltpu.VMEM((8, 8, W), jnp.float32),
        pltpu.VMEM((8, 8, WP), jnp.float32),
        pltpu.VMEM((8, 8, WP), jnp.float32),
        pltpu.SemaphoreType.DMA,
        pltpu.SemaphoreType.DMA,
        pltpu.SemaphoreType.DMA,
        pltpu.SemaphoreType.DMA,
    ],
    compiler_params=pltpu.CompilerParams(
        use_tc_tiling_on_sc=True, needs_layout_passes=False
    ),
)
def _pad_kernel(im_hbm, out_hbm, in0, in1, ot0, ot1, si0, si1, so0, so1):
    wid = lax.axis_index("s") * 2 + lax.axis_index("c")
    iota = lax.iota(jnp.int32, 16)

    cols = []
    for k in range(NCHUNK + 1):
        x = iota + (16 * k - 2)
        x = jnp.where(x < 0, -1 - x, x)
        x = jnp.where(x > W - 1, 2 * W - 1 - x, x)
        cols.append(x)
    dsts = [iota + 16 * k for k in range(NCHUNK)]
    tail_dst = jnp.minimum(16 * NCHUNK + iota, WP - 1)
    tail_msk = iota < WP - 16 * NCHUNK
    cvs = [jnp.full((16,), cl, jnp.int32) for cl in range(8)]

    def decode_a(u):
        q = u // NCB
        cb = u % NCB
        band = q % NBAND
        b = q // NBAND
        return b, 8 * band, 8 * band + 2, 8 * cb


    def start_load_a(u, buf, sem):
        b, iy, _, c0 = decode_a(u)
        pltpu.make_async_copy(
            im_hbm.at[b, pl.ds(c0, 8), pl.ds(iy, 8)], buf, sem
        ).start()

    def wait_load_a(buf, sem):
        pltpu.make_async_copy(
            im_hbm.at[0, pl.ds(0, 8), pl.ds(0, 8)], buf, sem
        ).wait()

    def start_store_a(u, obuf, sem):
        b, _, oy, c0 = decode_a(u)
        pltpu.make_async_copy(
            obuf, out_hbm.at[b, pl.ds(oy, 8), pl.ds(c0, 8)], sem
        ).start()

    def wait_store_a(obuf, sem):
        pltpu.make_async_copy(
            obuf, out_hbm.at[0, pl.ds(0, 8), pl.ds(0, 8)], sem
        ).wait()

    def gather_a(buf, obuf):
        def row_body(t, carry):
            tv = jnp.full((16,), t, jnp.int32)
            for cl in range(8):
                for k in range(NCHUNK):
                    v = plsc.load_gather(buf, [cvs[cl], tv, cols[k]])
                    plsc.store_scatter(obuf, [tv, cvs[cl], dsts[k]], v)
                v = plsc.load_gather(
                    buf, [cvs[cl], tv, cols[NCHUNK]], mask=tail_msk
                )
                plsc.store_scatter(
                    obuf, [tv, cvs[cl], tail_dst], v, mask=tail_msk
                )
            return carry

        lax.fori_loop(0, 8, row_body, 0)


    def decode_b(u):
        q = u // NCB
        cb = u % NCB
        top = q % 2
        b = q // 2
        iy = top * (H - 8)
        oy = top * (HP - 2)
        yin0 = 1 + top * 6
        return b, iy, oy, yin0, 8 * cb

    def start_load_b(u, buf, sem):
        b, iy, _, _, c0 = decode_b(u)
        pltpu.make_async_copy(
            im_hbm.at[b, pl.ds(c0, 8), pl.ds(iy, 8)], buf, sem
        ).start()

    def start_store_b(u, obuf, sem):
        b, _, oy, _, c0 = decode_b(u)
        pltpu.make_async_copy(
            obuf.at[pl.ds(0, 2)], out_hbm.at[b, pl.ds(oy, 2), pl.ds(c0, 8)], sem
        ).start()

    def wait_store_b(obuf, sem):
        pltpu.make_async_copy(
            obuf.at[pl.ds(0, 2)], out_hbm.at[0, pl.ds(0, 2), pl.ds(0, 8)], sem
        ).wait()

    def gather_b(u, buf, obuf):
        _, _, _, yin0, _ = decode_b(u)
        for t in range(2):
            tv = jnp.full((16,), t, jnp.int32)
            yv = jnp.full((16,), yin0 - t, jnp.int32)
            for cl in range(8):
                for k in range(NCHUNK):
                    v = plsc.load_gather(buf, [cvs[cl], yv, cols[k]])
                    plsc.store_scatter(obuf, [tv, cvs[cl], dsts[k]], v)
                v = plsc.load_gather(
                    buf, [cvs[cl], yv, cols[NCHUNK]], mask=tail_msk
                )
                plsc.store_scatter(
                    obuf, [tv, cvs[cl], tail_dst], v, mask=tail_msk
                )

    a0 = wid * PER_WA
    start_load_a(a0, in0, si0)

    def body_a(j, carry):
        u0 = a0 + 2 * j
        wait_load_a(in0, si0)
        pl.when(j > 0)(lambda: wait_store_a(ot1, so1))
        start_load_a(u0 + 1, in1, si1)
        gather_a(in0, ot0)
        start_store_a(u0, ot0, so0)
        wait_load_a(in1, si1)
        gather_a(in1, ot1)
        wait_store_a(ot0, so0)
        pl.when(j < PER_WA // 2 - 1)(lambda: start_load_a(u0 + 2, in0, si0))
        start_store_a(u0 + 1, ot1, so1)
        return carry

    lax.fori_loop(0, PER_WA // 2, body_a, 0)
    wait_store_a(ot1, so1)

    b0 = wid * PER_WB
    start_load_b(b0, in0, si0)

    def body_b(j, carry):
        u0 = b0 + 2 * j
        wait_load_a(in0, si0)
        pl.when(j > 0)(lambda: wait_store_b(ot1, so1))
        start_load_b(u0 + 1, in1, si1)
        gather_b(u0, in0, ot0)
        start_store_b(u0, ot0, so0)
        wait_load_a(in1, si1)
        gather_b(u0 + 1, in1, ot1)
        wait_store_b(ot0, so0)
        pl.when(j < PER_WB // 2 - 1)(lambda: start_load_b(u0 + 2, in0, si0))
        start_store_b(u0 + 1, ot1, so1)
        return carry

    lax.fori_loop(0, PER_WB // 2, body_b, 0)
    wait_store_b(ot1, so1)


def kernel(im):
    out = _pad_kernel(im)
    return out.transpose(0, 2, 1, 3)


# speedup vs baseline: 3.3812x; 1.4999x over previous
import functools

import jax
import jax.numpy as jnp
from jax import lax
from jax.experimental import pallas as pl
from jax.experimental.pallas import tpu as pltpu
from jax.experimental.pallas import tpu_sc as plsc

B = 16
C = 96
H = 224
W = 224
HP = H + 4
WP = W + 4
NWORK = 32
NCHUNK = WP // 16

NBAND = H // 8
NCB = C // 8
NA = B * NBAND * NCB
PER_WA = NA // NWORK
NB = B * NCB * 2
PER_WB = NB // NWORK

_MESH = plsc.VectorSubcoreMesh(
    core_axis_name="c", subcore_axis_name="s", num_cores=2, num_subcores=16
)


@functools.partial(
    pl.kernel,
    out_type=jax.ShapeDtypeStruct((B, HP, C, WP), jnp.float32),
    mesh=_MESH,
    scratch_types=[
        pltpu.VMEM((8, 8, W), jnp.float32),
        pltpu.VMEM((8, 8, W), jnp.float32),
        pltpu.VMEM((8, 8, WP), jnp.float32),
        pltpu.VMEM((8, 8, WP), jnp.float32),
        pltpu.SemaphoreType.DMA,
        pltpu.SemaphoreType.DMA,
        pltpu.SemaphoreType.DMA,
        pltpu.SemaphoreType.DMA,
    ],
    compiler_params=pltpu.CompilerParams(
        use_tc_tiling_on_sc=True, needs_layout_passes=False
    ),
)
def _pad_kernel(im_hbm, out_hbm, in0, in1, ot0, ot1, si0, si1, so0, so1):
    wid = lax.axis_index("s") * 2 + lax.axis_index("c")
    iota = lax.iota(jnp.int32, 16)

    cols = []
    for k in range(NCHUNK + 1):
        x = iota + (16 * k - 2)
        x = jnp.where(x < 0, -1 - x, x)
        x = jnp.where(x > W - 1, 2 * W - 1 - x, x)
        cols.append(x)
    dsts = [iota + 16 * k for k in range(NCHUNK)]
    tail_dst = jnp.minimum(16 * NCHUNK + iota, WP - 1)
    tail_msk = iota < WP - 16 * NCHUNK
    cvs = [jnp.full((16,), cl, jnp.int32) for cl in range(8)]

    def decode_a(u):
        q = u // NCB
        cb = u % NCB
        band = q % NBAND
        b = q // NBAND
        return b, 8 * band, 8 * band + 2, 8 * cb


    def start_load_a(u, buf, sem):
        b, iy, _, c0 = decode_a(u)
        pltpu.make_async_copy(
            im_hbm.at[b, pl.ds(c0, 8), pl.ds(iy, 8)], buf, sem
        ).start()

    def wait_load_a(buf, sem):
        pltpu.make_async_copy(
            im_hbm.at[0, pl.ds(0, 8), pl.ds(0, 8)], buf, sem
        ).wait()

    def start_store_a(u, obuf, sem):
        b, _, oy, c0 = decode_a(u)
        pltpu.make_async_copy(
            obuf, out_hbm.at[b, pl.ds(oy, 8), pl.ds(c0, 8)], sem
        ).start()

    def wait_store_a(obuf, sem):
        pltpu.make_async_copy(
            obuf, out_hbm.at[0, pl.ds(0, 8), pl.ds(0, 8)], sem
        ).wait()

    def gather_a(buf, obuf):
        @plsc.parallel_loop(0, 8, step=1, unroll=2)
        def row_body(t):
            tv = jnp.full((16,), t, jnp.int32)
            for cl in range(8):
                for k in range(NCHUNK):
                    v = plsc.load_gather(buf, [cvs[cl], tv, cols[k]])
                    plsc.store_scatter(obuf, [tv, cvs[cl], dsts[k]], v)
                v = plsc.load_gather(
                    buf, [cvs[cl], tv, cols[NCHUNK]], mask=tail_msk
                )
                plsc.store_scatter(
                    obuf, [tv, cvs[cl], tail_dst], v, mask=tail_msk
                )


    def decode_b(u):
        q = u // NCB
        cb = u % NCB
        top = q % 2
        b = q // 2
        iy = top * (H - 8)
        oy = top * (HP - 2)
        yin0 = 1 + top * 6
        return b, iy, oy, yin0, 8 * cb

    def start_load_b(u, buf, sem):
        b, iy, _, _, c0 = decode_b(u)
        pltpu.make_async_copy(
            im_hbm.at[b, pl.ds(c0, 8), pl.ds(iy, 8)], buf, sem
        ).start()

    def start_store_b(u, obuf, sem):
        b, _, oy, _, c0 = decode_b(u)
        pltpu.make_async_copy(
            obuf.at[pl.ds(0, 2)], out_hbm.at[b, pl.ds(oy, 2), pl.ds(c0, 8)], sem
        ).start()

    def wait_store_b(obuf, sem):
        pltpu.make_async_copy(
            obuf.at[pl.ds(0, 2)], out_hbm.at[0, pl.ds(0, 2), pl.ds(0, 8)], sem
        ).wait()

    def gather_b(u, buf, obuf):
        _, _, _, yin0, _ = decode_b(u)
        for t in range(2):
            tv = jnp.full((16,), t, jnp.int32)
            yv = jnp.full((16,), yin0 - t, jnp.int32)
            for cl in range(8):
                for k in range(NCHUNK):
                    v = plsc.load_gather(buf, [cvs[cl], yv, cols[k]])
                    plsc.store_scatter(obuf, [tv, cvs[cl], dsts[k]], v)
                v = plsc.load_gather(
                    buf, [cvs[cl], yv, cols[NCHUNK]], mask=tail_msk
                )
                plsc.store_scatter(
                    obuf, [tv, cvs[cl], tail_dst], v, mask=tail_msk
                )

    a0 = wid * PER_WA
    start_load_a(a0, in0, si0)

    def body_a(j, carry):
        u0 = a0 + 2 * j
        wait_load_a(in0, si0)
        pl.when(j > 0)(lambda: wait_store_a(ot1, so1))
        start_load_a(u0 + 1, in1, si1)
        gather_a(in0, ot0)
        start_store_a(u0, ot0, so0)
        wait_load_a(in1, si1)
        gather_a(in1, ot1)
        wait_store_a(ot0, so0)
        pl.when(j < PER_WA // 2 - 1)(lambda: start_load_a(u0 + 2, in0, si0))
        start_store_a(u0 + 1, ot1, so1)
        return carry

    lax.fori_loop(0, PER_WA // 2, body_a, 0)
    wait_store_a(ot1, so1)

    b0 = wid * PER_WB
    start_load_b(b0, in0, si0)

    def body_b(j, carry):
        u0 = b0 + 2 * j
        wait_load_a(in0, si0)
        pl.when(j > 0)(lambda: wait_store_b(ot1, so1))
        start_load_b(u0 + 1, in1, si1)
        gather_b(u0, in0, ot0)
        start_store_b(u0, ot0, so0)
        wait_load_a(in1, si1)
        gather_b(u0 + 1, in1, ot1)
        wait_store_b(ot0, so0)
        pl.when(j < PER_WB // 2 - 1)(lambda: start_load_b(u0 + 2, in0, si0))
        start_store_b(u0 + 1, ot1, so1)
        return carry

    lax.fori_loop(0, PER_WB // 2, body_b, 0)
    wait_store_b(ot1, so1)


def kernel(im):
    out = _pad_kernel(im)
    return out.transpose(0, 2, 1, 3)
